# baseline (speedup 1.0000x reference)
import functools

import jax
import jax.numpy as jnp
from jax import lax
from jax.experimental import pallas as pl
from jax.experimental.pallas import tpu as pltpu
from jax.experimental.pallas import tpu_sc as plsc

BATCH = 16384
NUM_FIELDS = 26
EMBED_DIM = 16
FIELD_SIZE = 100000

TOTAL_ROWS = NUM_FIELDS * FIELD_SIZE
DETILE_W = 32768
W_SHIFT = 15
NBLK = 80
OCT_STRIDE = NBLK * 8 * DETILE_W
SUB_STRIDE = DETILE_W
GLEN = 20720704
EMB_FLAT = 2 * OCT_STRIDE

NC, NS, LANES = 2, 16, 16
NW = NC * NS
SAMPLES_PER_W = BATCH // NW
CHUNK = 128
NCHUNKS = SAMPLES_PER_W // CHUNK
NBLOCKS = CHUNK // LANES


def _fm_body(idx_hbm, emb_hbm, fc_hbm, bias_hbm, out_hbm,
             idx_v, idxp_v, rows_v, fc_v, out_v, bias_v, sem_e, sem_f):
    wid = lax.axis_index("s") * NC + lax.axis_index("c")
    wbase = wid * SAMPLES_PER_W

    pltpu.sync_copy(bias_hbm, bias_v)
    bvec = bias_v[...]
    zeros16 = jnp.zeros((LANES,), jnp.float32)

    def chunk_body(c, carry):
        sbase = wbase + c * CHUNK
        for f in range(NUM_FIELDS):
            pltpu.sync_copy(idx_hbm.at[f, pl.ds(sbase, CHUNK)],
                            idx_v.at[pl.ds(f * CHUNK, CHUNK)])

        def conv_body(k, kcarry):
            r = idx_v[pl.ds(k * LANES, LANES)]
            idxp_v[pl.ds(k * LANES, LANES)] = (
                ((r >> W_SHIFT) << (W_SHIFT + 3)) + (r & (DETILE_W - 1)))
            return kcarry

        lax.fori_loop(0, NUM_FIELDS * CHUNK // LANES, conv_body, 0)

        copies = [pltpu.async_copy(fc_hbm.at[idx_v], fc_v, sem_f)]
        for d in range(EMBED_DIM):
            off = (d // 8) * OCT_STRIDE + (d % 8) * SUB_STRIDE
            copies.append(pltpu.async_copy(
                emb_hbm.at[pl.ds(off, GLEN)].at[idxp_v],
                rows_v.at[d], sem_e))
        for cp in copies:
            cp.wait()

        def block_body(b, bcarry):
            col = b * LANES
            acc = zeros16
            for d in range(EMBED_DIM):
                s = zeros16
                q = zeros16
                for f in range(NUM_FIELDS):
                    v = rows_v[d, pl.ds(f * CHUNK + col, LANES)]
                    s = s + v
                    q = q + v * v
                acc = acc + (s * s - q)
            lin = zeros16
            for f in range(NUM_FIELDS):
                lin = lin + fc_v[pl.ds(f * CHUNK + col, LANES)]
            out_v[pl.ds(col, LANES)] = 0.5 * acc + lin + bvec
            return bcarry

        lax.fori_loop(0, NBLOCKS, block_body, 0)
        pltpu.sync_copy(out_v, out_hbm.at[pl.ds(sbase, CHUNK)])
        return carry

    lax.fori_loop(0, NCHUNKS, chunk_body, 0)


_fm_kernel = functools.partial(
    pl.kernel,
    out_type=jax.ShapeDtypeStruct((BATCH,), jnp.float32),
    mesh=plsc.VectorSubcoreMesh(core_axis_name="c", subcore_axis_name="s"),
    compiler_params=pltpu.CompilerParams(needs_layout_passes=False,
                                         use_tc_tiling_on_sc=False),
    scratch_types=[
        pltpu.VMEM((NUM_FIELDS * CHUNK,), jnp.int32),
        pltpu.VMEM((NUM_FIELDS * CHUNK,), jnp.int32),
        pltpu.VMEM((EMBED_DIM, NUM_FIELDS * CHUNK), jnp.float32),
        pltpu.VMEM((NUM_FIELDS * CHUNK,), jnp.float32),
        pltpu.VMEM((CHUNK,), jnp.float32),
        pltpu.VMEM((LANES,), jnp.float32),
        pltpu.SemaphoreType.DMA,
        pltpu.SemaphoreType.DMA,
    ],
)(_fm_body)


def _detile_body(in_ref, fc_ref, out_ref, fcout_ref):
    out_ref[...] = in_ref[...].reshape(8 * DETILE_W)
    fcout_ref[...] = fc_ref[...].reshape(DETILE_W)


_detile = pl.pallas_call(
    _detile_body,
    out_shape=(jax.ShapeDtypeStruct((EMB_FLAT,), jnp.float32),
               jax.ShapeDtypeStruct((NBLK * DETILE_W,), jnp.float32)),
    grid=(2, NBLK),
    in_specs=[pl.BlockSpec((8, DETILE_W), lambda i, j: (i, j)),
              pl.BlockSpec((DETILE_W, 1), lambda i, j: (j, 0))],
    out_specs=(pl.BlockSpec((8 * DETILE_W,), lambda i, j: (i * NBLK + j)),
               pl.BlockSpec((DETILE_W,), lambda i, j: (j,))),
)


def kernel(x, emb_table, fc_weight, bias):
    offs = jnp.arange(NUM_FIELDS, dtype=jnp.int32) * FIELD_SIZE
    idx_fm = x.T + offs[:, None]
    emb_planes, fc1d = _detile(emb_table.T, fc_weight)
    bias16 = jnp.broadcast_to(bias, (LANES,))
    return _fm_kernel(idx_fm, emb_planes, fc1d, bias16)


# speedup vs baseline: 5.3254x; 5.3254x over previous
import functools

import jax
import jax.numpy as jnp
from jax import lax
from jax.experimental import pallas as pl
from jax.experimental.pallas import tpu as pltpu
from jax.experimental.pallas import tpu_sc as plsc

BATCH = 16384
NUM_FIELDS = 26
EMBED_DIM = 16
FIELD_SIZE = 100000

TOTAL_ROWS = NUM_FIELDS * FIELD_SIZE
DETILE_W = 32768
W_SHIFT = 15
NBLK = 80
OCT_STRIDE = NBLK * 8 * DETILE_W
SUB_STRIDE = DETILE_W
GLEN = 20720704
EMB_FLAT = 2 * OCT_STRIDE

NC, NS, LANES = 2, 16, 16
NW = NC * NS
SAMPLES_PER_W = BATCH // NW
CHUNK = 64
NCHUNKS = SAMPLES_PER_W // CHUNK
NBLOCKS = CHUNK // LANES
IDXC = NUM_FIELDS * CHUNK


def _fm_body(idx_hbm, emb_hbm, fc_hbm, bias_hbm, out_hbm,
             idx2_v, idxl_v, idxp_v, rows_v, fc_v, out_v, bias_v,
             sem_e0, sem_e1, sem_f0, sem_f1):
    wid = lax.axis_index("s") * NC + lax.axis_index("c")
    wbase = wid * SAMPLES_PER_W
    sems_e = (sem_e0, sem_e1)
    sems_f = (sem_f0, sem_f1)

    pltpu.sync_copy(bias_hbm, bias_v)
    pltpu.sync_copy(idx_hbm.at[:, pl.ds(wbase, SAMPLES_PER_W)], idx2_v)
    bvec = bias_v[...]
    zeros16 = jnp.zeros((LANES,), jnp.float32)

    def repack(c, s):
        def rep_body(f, fcarry):
            for k in range(CHUNK // LANES):
                r = idx2_v[f, pl.ds(c * CHUNK + k * LANES, LANES)]
                dst = f * CHUNK + k * LANES
                idxl_v[s, pl.ds(dst, LANES)] = r
                idxp_v[s, pl.ds(dst, LANES)] = (
                    ((r >> W_SHIFT) << (W_SHIFT + 3)) + (r & (DETILE_W - 1)))
            return fcarry

        lax.fori_loop(0, NUM_FIELDS, rep_body, 0)

    def fire(s):
        copies = [pltpu.async_copy(fc_hbm.at[idxl_v.at[s]],
                                   fc_v.at[s], sems_f[s])]
        for d in range(EMBED_DIM):
            off = (d // 8) * OCT_STRIDE + (d % 8) * SUB_STRIDE
            copies.append(pltpu.async_copy(
                emb_hbm.at[pl.ds(off, GLEN)].at[idxp_v.at[s]],
                rows_v.at[s, d], sems_e[s]))
        return copies

    def compute(c, s):
        def block_body(b, bcarry):
            col = b * LANES

            def f_body(f, carry):
                accs = list(carry)
                lin = accs[-1]
                base = f * CHUNK + col
                for d in range(EMBED_DIM):
                    v = rows_v[s, d, pl.ds(base, LANES)]
                    accs[2 * d] = accs[2 * d] + v
                    accs[2 * d + 1] = accs[2 * d + 1] + v * v
                accs[-1] = lin + fc_v[s, pl.ds(base, LANES)]
                return tuple(accs)

            init = tuple([zeros16] * (2 * EMBED_DIM + 1))
            res = lax.fori_loop(0, NUM_FIELDS, f_body, init)
            acc = zeros16
            for d in range(EMBED_DIM):
                acc = acc + (res[2 * d] * res[2 * d] - res[2 * d + 1])
            out_v[pl.ds(c * CHUNK + col, LANES)] = (
                0.5 * acc + res[-1] + bvec)
            return bcarry

        lax.fori_loop(0, NBLOCKS, block_body, 0)

    repack(0, 0)
    inflight = {0: fire(0)}
    for c in range(NCHUNKS):
        s = c % 2
        if c + 1 < NCHUNKS:
            repack(c + 1, 1 - s)
            inflight[1 - s] = fire(1 - s)
        for cp in inflight.pop(s):
            cp.wait()
        compute(c, s)

    pltpu.sync_copy(out_v, out_hbm.at[pl.ds(wbase, SAMPLES_PER_W)])


_fm_kernel = functools.partial(
    pl.kernel,
    out_type=jax.ShapeDtypeStruct((BATCH,), jnp.float32),
    mesh=plsc.VectorSubcoreMesh(core_axis_name="c", subcore_axis_name="s"),
    compiler_params=pltpu.CompilerParams(needs_layout_passes=False,
                                         use_tc_tiling_on_sc=False),
    scratch_types=[
        pltpu.VMEM((NUM_FIELDS, SAMPLES_PER_W), jnp.int32),
        pltpu.VMEM((2, IDXC), jnp.int32),
        pltpu.VMEM((2, IDXC), jnp.int32),
        pltpu.VMEM((2, EMBED_DIM, IDXC), jnp.float32),
        pltpu.VMEM((2, IDXC), jnp.float32),
        pltpu.VMEM((SAMPLES_PER_W,), jnp.float32),
        pltpu.VMEM((LANES,), jnp.float32),
        pltpu.SemaphoreType.DMA,
        pltpu.SemaphoreType.DMA,
        pltpu.SemaphoreType.DMA,
        pltpu.SemaphoreType.DMA,
    ],
)(_fm_body)


def _detile_body(in_ref, out_ref):
    out_ref[...] = in_ref[...].reshape(8 * DETILE_W)


_detile = pl.pallas_call(
    _detile_body,
    out_shape=jax.ShapeDtypeStruct((EMB_FLAT,), jnp.float32),
    grid=(2, NBLK),
    in_specs=[pl.BlockSpec((8, DETILE_W), lambda i, j: (i, j))],
    out_specs=pl.BlockSpec((8 * DETILE_W,), lambda i, j: (i * NBLK + j)),
)


def kernel(x, emb_table, fc_weight, bias):
    offs = jnp.arange(NUM_FIELDS, dtype=jnp.int32) * FIELD_SIZE
    idx_fm = x.T + offs[:, None]
    emb_planes = _detile(emb_table.T)
    fc1d = fc_weight.T.reshape(-1)
    bias16 = jnp.broadcast_to(bias, (LANES,))
    return _fm_kernel(idx_fm, emb_planes, fc1d, bias16)
